# baseline (speedup 1.0000x reference)
import functools

import jax
import jax.numpy as jnp
from jax import lax
from jax.experimental import pallas as pl
from jax.experimental.pallas import tpu as pltpu
from jax.experimental.pallas import tpu_sc as plsc

VOCAB = 1000000
EMBED = 64
B_ROWS = 4096
B_COLS = 50
CHUNK = 128

_info = plsc.get_sparse_core_info()
NC, NS = _info.num_cores, _info.num_subcores
NW = NC * NS
NBUF = 10


def _make_kernel():
    mesh = plsc.VectorSubcoreMesh(core_axis_name="c", subcore_axis_name="s")

    @functools.partial(
        pl.kernel,
        mesh=mesh,
        compiler_params=pltpu.CompilerParams(use_tc_tiling_on_sc=False),
        out_type=jax.ShapeDtypeStruct((B_ROWS, B_COLS, EMBED), jnp.float32),
        scratch_types=[
            pltpu.VMEM((B_COLS, CHUNK), jnp.int32),
            pltpu.VMEM((NBUF, CHUNK, EMBED), jnp.float32),
            [pltpu.SemaphoreType.DMA] * NBUF,
        ],
    )
    def k(idx_hbm, table_hbm, out_hbm, idx_v, rows_v, sems):
        wid = lax.axis_index("s") * NC + lax.axis_index("c")
        b0 = pl.multiple_of(wid * CHUNK, CHUNK)

        pltpu.sync_copy(idx_hbm.at[:, pl.ds(b0, CHUNK)], idx_v)

        for b in range(NBUF):
            pltpu.async_copy(table_hbm.at[idx_v.at[b]], rows_v.at[b], sems[b])

        @pl.loop(0, B_COLS, step=NBUF)
        def _ring(s0):
            for b in range(NBUF):
                s = s0 + b
                pltpu.make_async_copy(table_hbm.at[idx_v.at[s]], rows_v.at[b],
                                      sems[b]).wait()
                pltpu.sync_copy(rows_v.at[b],
                                out_hbm.at[pl.ds(b0, CHUNK), s])
                nxt = s + NBUF

                @pl.when(nxt < B_COLS)
                def _():
                    pltpu.async_copy(table_hbm.at[idx_v.at[nxt]], rows_v.at[b],
                                     sems[b])

    return k


_kernel_call = _make_kernel()


def kernel(inputs, embeddings):
    idx_t = jnp.transpose(inputs.astype(jnp.int32))
    return _kernel_call(idx_t, embeddings)


# speedup vs baseline: 1.0164x; 1.0164x over previous
import functools

import jax
import jax.numpy as jnp
from jax import lax
from jax.experimental import pallas as pl
from jax.experimental.pallas import tpu as pltpu
from jax.experimental.pallas import tpu_sc as plsc

VOCAB = 1000000
EMBED = 64
B_ROWS = 4096
B_COLS = 50
CHUNK = 128

_info = plsc.get_sparse_core_info()
NC, NS = _info.num_cores, _info.num_subcores
NW = NC * NS
NBUF = 10


def _make_kernel():
    mesh = plsc.VectorSubcoreMesh(core_axis_name="c", subcore_axis_name="s")

    @functools.partial(
        pl.kernel,
        mesh=mesh,
        compiler_params=pltpu.CompilerParams(use_tc_tiling_on_sc=False),
        out_type=jax.ShapeDtypeStruct((B_COLS, B_ROWS, EMBED), jnp.float32),
        scratch_types=[
            pltpu.VMEM((B_COLS, CHUNK), jnp.int32),
            pltpu.VMEM((NBUF, CHUNK, EMBED), jnp.float32),
            [pltpu.SemaphoreType.DMA] * NBUF,
        ],
    )
    def k(idx_hbm, table_hbm, out_hbm, idx_v, rows_v, sems):
        wid = lax.axis_index("s") * NC + lax.axis_index("c")
        b0 = pl.multiple_of(wid * CHUNK, CHUNK)

        pltpu.sync_copy(idx_hbm.at[:, pl.ds(b0, CHUNK)], idx_v)

        for b in range(NBUF):
            pltpu.async_copy(table_hbm.at[idx_v.at[b]], rows_v.at[b], sems[b])

        @pl.loop(0, B_COLS, step=NBUF)
        def _ring(s0):
            for b in range(NBUF):
                s = s0 + b
                pltpu.make_async_copy(table_hbm.at[idx_v.at[s]], rows_v.at[b],
                                      sems[b]).wait()
                pltpu.sync_copy(rows_v.at[b],
                                out_hbm.at[s].at[pl.ds(b0, CHUNK)])
                nxt = s + NBUF

                @pl.when(nxt < B_COLS)
                def _():
                    pltpu.async_copy(table_hbm.at[idx_v.at[nxt]], rows_v.at[b],
                                     sems[b])

    return k


_kernel_call = _make_kernel()


def kernel(inputs, embeddings):
    idx_t = jnp.transpose(inputs.astype(jnp.int32))
    out_d = _kernel_call(idx_t, embeddings)
    return jnp.transpose(out_d, (1, 0, 2))
